# baseline (speedup 1.0000x reference)
import functools

import jax
from jax import lax
import jax.numpy as jnp
from jax.experimental import pallas as pl
from jax.experimental.pallas import tpu as pltpu
from jax.experimental.pallas import tpu_sc as plsc

_NUM_SUBCORES = 16


def kernel(x, table):
    b, l = x.shape
    _, d = table.shape
    nw = _NUM_SUBCORES
    b_per_w = b // nw
    chunk = 4
    nchunks = b_per_w // chunk
    assert b_per_w * nw == b and chunk * nchunks == b_per_w and nchunks % 2 == 0

    mesh = plsc.VectorSubcoreMesh(
        core_axis_name="c", subcore_axis_name="s", num_cores=1
    )

    @functools.partial(
        pl.kernel,
        mesh=mesh,
        out_type=jax.ShapeDtypeStruct((b, l, d), table.dtype),
        scratch_types=[
            pltpu.VMEM((b_per_w, l), jnp.int32),
            pltpu.VMEM((chunk, l, d), table.dtype),
            pltpu.VMEM((chunk, l, d), table.dtype),
            pltpu.SemaphoreType.DMA,
            pltpu.SemaphoreType.DMA,
            pltpu.SemaphoreType.DMA,
            pltpu.SemaphoreType.DMA,
        ],
    )
    def gather_kernel(tab_hbm, x_hbm, o_hbm, idx_v, buf0, buf1, g0, g1, o0, o1):
        wid = lax.axis_index("s")
        base = wid * b_per_w
        pltpu.sync_copy(x_hbm.at[pl.ds(base, b_per_w)], idx_v)

        bufs = (buf0, buf1)
        gsems = (g0, g1)
        osems = (o0, o1)

        @pl.loop(0, nchunks, step=2)
        def _(g):
            for bi in range(2):
                buf, gsem, osem = bufs[bi], gsems[bi], osems[bi]
                gg = g + bi

                @pl.when(gg >= 2)
                def _():
                    pltpu.make_async_copy(
                        buf, o_hbm.at[pl.ds(base, chunk)], osem
                    ).wait()

                copies = [
                    pltpu.async_copy(
                        tab_hbm.at[idx_v.at[gg * chunk + r]], buf.at[r], gsem
                    )
                    for r in range(chunk)
                ]
                for cp in copies:
                    cp.wait()
                pltpu.async_copy(
                    buf, o_hbm.at[pl.ds(base + gg * chunk, chunk)], osem
                )

        for bi in range(2):
            pltpu.make_async_copy(
                bufs[bi], o_hbm.at[pl.ds(base, chunk)], osems[bi]
            ).wait()

    return gather_kernel(table, x)


# speedup vs baseline: 1.0820x; 1.0820x over previous
import functools

import jax
from jax import lax
import jax.numpy as jnp
from jax.experimental import pallas as pl
from jax.experimental.pallas import tpu as pltpu
from jax.experimental.pallas import tpu_sc as plsc

_NUM_SUBCORES = 16


def kernel(x, table):
    b, l = x.shape
    _, d = table.shape
    nw = _NUM_SUBCORES
    b_per_w = b // nw
    chunk = 8
    phase_seqs = 64
    phase_chunks = phase_seqs // chunk
    nphases = b_per_w // phase_seqs
    assert b_per_w * nw == b
    assert phase_seqs * nphases == b_per_w and chunk * phase_chunks == phase_seqs
    assert phase_chunks % 2 == 0

    mesh = plsc.VectorSubcoreMesh(
        core_axis_name="c", subcore_axis_name="s", num_cores=1
    )

    @functools.partial(
        pl.kernel,
        mesh=mesh,
        out_type=jax.ShapeDtypeStruct((b, l, d), table.dtype),
        scratch_types=[
            pltpu.VMEM((phase_seqs, l), jnp.int32),
            pltpu.VMEM((chunk, l, d), table.dtype),
            pltpu.VMEM((chunk, l, d), table.dtype),
            pltpu.SemaphoreType.DMA,
            pltpu.SemaphoreType.DMA,
            pltpu.SemaphoreType.DMA,
            pltpu.SemaphoreType.DMA,
        ],
    )
    def gather_kernel(tab_hbm, x_hbm, o_hbm, idx_v, buf0, buf1, g0, g1, o0, o1):
        wid = lax.axis_index("s")
        base = wid * b_per_w

        bufs = (buf0, buf1)
        gsems = (g0, g1)
        osems = (o0, o1)

        @pl.loop(0, nphases)
        def _(p):
            pbase = base + p * phase_seqs
            pltpu.sync_copy(x_hbm.at[pl.ds(pbase, phase_seqs)], idx_v)

            @pl.loop(0, phase_chunks, step=2)
            def _(c):
                for bi in range(2):
                    buf, gsem, osem = bufs[bi], gsems[bi], osems[bi]
                    cc = c + bi
                    gg = p * phase_chunks + cc

                    @pl.when(gg >= 2)
                    def _():
                        pltpu.make_async_copy(
                            buf, o_hbm.at[pl.ds(base, chunk)], osem
                        ).wait()

                    copies = [
                        pltpu.async_copy(
                            tab_hbm.at[idx_v.at[cc * chunk + r]], buf.at[r], gsem
                        )
                        for r in range(chunk)
                    ]
                    for cp in copies:
                        cp.wait()
                    pltpu.async_copy(
                        buf, o_hbm.at[pl.ds(pbase + cc * chunk, chunk)], osem
                    )

        for bi in range(2):
            pltpu.make_async_copy(
                bufs[bi], o_hbm.at[pl.ds(base, chunk)], osems[bi]
            ).wait()

    return gather_kernel(table, x)


# speedup vs baseline: 1.3065x; 1.2075x over previous
import functools

import jax
from jax import lax
import jax.numpy as jnp
from jax.experimental import pallas as pl
from jax.experimental.pallas import tpu as pltpu
from jax.experimental.pallas import tpu_sc as plsc

_NUM_CORES = 2
_NUM_SUBCORES = 16


def kernel(x, table):
    b, l = x.shape
    _, d = table.shape
    nw = _NUM_CORES * _NUM_SUBCORES
    b_per_w = b // nw
    chunk = 8
    nchunks = b_per_w // chunk
    assert b_per_w * nw == b and chunk * nchunks == b_per_w and nchunks % 2 == 0

    mesh = plsc.VectorSubcoreMesh(core_axis_name="c", subcore_axis_name="s")

    @functools.partial(
        pl.kernel,
        mesh=mesh,
        out_type=jax.ShapeDtypeStruct((b, l, d), table.dtype),
        scratch_types=[
            pltpu.VMEM((b_per_w, l), jnp.int32),
            pltpu.VMEM((chunk, l, d), table.dtype),
            pltpu.VMEM((chunk, l, d), table.dtype),
            pltpu.SemaphoreType.DMA,
            pltpu.SemaphoreType.DMA,
            pltpu.SemaphoreType.DMA,
            pltpu.SemaphoreType.DMA,
        ],
    )
    def gather_kernel(tab_hbm, x_hbm, o_hbm, idx_v, buf0, buf1, g0, g1, o0, o1):
        wid = lax.axis_index("c") * _NUM_SUBCORES + lax.axis_index("s")
        base = wid * b_per_w
        pltpu.sync_copy(x_hbm.at[pl.ds(base, b_per_w)], idx_v)

        bufs = (buf0, buf1)
        gsems = (g0, g1)
        osems = (o0, o1)

        @pl.loop(0, nchunks, step=2)
        def _(g):
            for bi in range(2):
                buf, gsem, osem = bufs[bi], gsems[bi], osems[bi]
                gg = g + bi

                @pl.when(gg >= 2)
                def _():
                    pltpu.make_async_copy(
                        buf, o_hbm.at[pl.ds(base, chunk)], osem
                    ).wait()

                copies = [
                    pltpu.async_copy(
                        tab_hbm.at[idx_v.at[gg * chunk + r]], buf.at[r], gsem
                    )
                    for r in range(chunk)
                ]
                for cp in copies:
                    cp.wait()
                pltpu.async_copy(
                    buf, o_hbm.at[pl.ds(base + gg * chunk, chunk)], osem
                )

        for bi in range(2):
            pltpu.make_async_copy(
                bufs[bi], o_hbm.at[pl.ds(base, chunk)], osems[bi]
            ).wait()

    return gather_kernel(table, x)
